# baseline (speedup 1.0000x reference)
import functools

import jax
import jax.numpy as jnp
from jax import lax
from jax.experimental import pallas as pl
from jax.experimental.pallas import tpu as pltpu
from jax.experimental.pallas import tpu_sc as plsc

_K = 16
_SHIFT = _K - 1
_NC, _NS = 2, 16
_CHUNK = 32


def kernel(x):
    B, L, D = x.shape
    nw = _NC * _NS
    rows_per_w = (B * L) // nw
    workers_per_b = L // rows_per_w
    nchunks = rows_per_w // _CHUNK

    x2 = x.reshape(B * L, D)
    mesh = plsc.VectorSubcoreMesh(core_axis_name="c", subcore_axis_name="s")

    @functools.partial(
        pl.kernel,
        mesh=mesh,
        out_type=jax.ShapeDtypeStruct((B, L, 2, D), x.dtype),
        scratch_types=[
            pltpu.VMEM((_CHUNK, D), x.dtype),
            pltpu.VMEM((_CHUNK, D), x.dtype),
            pltpu.VMEM((_CHUNK,), jnp.int32),
            pltpu.VMEM((_SHIFT, D), x.dtype),
            pltpu.SemaphoreType.DMA,
            pltpu.SemaphoreType.DMA,
            pltpu.SemaphoreType.DMA,
            pltpu.SemaphoreType.DMA,
        ],
    )
    def span_sc(x_hbm, out_hbm, buf0, buf1, idx_v, zbuf, is0, is1, os0, os1):
        wid = lax.axis_index("s") * _NC + lax.axis_index("c")
        b = wid // workers_per_b
        s = (wid % workers_per_b) * rows_per_w
        base = b * L + s
        bufs = (buf0, buf1)
        ins = (is0, is1)
        outs = (os0, os1)

        def in_copy(g):
            return pltpu.make_async_copy(
                x_hbm.at[pl.ds(base + g * _CHUNK, _CHUNK)],
                bufs[g % 2],
                ins[g % 2],
            )

        def out0_copy(g):
            return pltpu.make_async_copy(
                bufs[g % 2],
                out_hbm.at[b, pl.ds(s + g * _CHUNK, _CHUNK), 0],
                outs[g % 2],
            )

        def out1_copy(g):
            return pltpu.make_async_copy(
                bufs[g % 2],
                out_hbm.at[b, pl.ds(s + g * _CHUNK - _SHIFT, _CHUNK), 1],
                outs[g % 2],
            )

        in_copy(0).start()
        in_copy(1).start()
        for g in range(nchunks):
            in_copy(g).wait()
            out0_copy(g).start()
            o1_ok = jnp.logical_or(g > 0, s > 0)

            @pl.when(o1_ok)
            def _():
                out1_copy(g).start()

            out0_copy(g).wait()

            @pl.when(o1_ok)
            def _():
                out1_copy(g).wait()

            if g + 2 < nchunks:
                in_copy(g + 2).start()

        @pl.when(s == 0)
        def _():
            for j in range(_CHUNK // 16):
                idx_v[pl.ds(j * 16, 16)] = (
                    lax.iota(jnp.int32, 16) + (b * L + _SHIFT + j * 16)
                )
            pltpu.async_copy(x_hbm.at[idx_v], buf0, is0).wait()
            pltpu.sync_copy(buf0, out_hbm.at[b, pl.ds(0, _CHUNK), 1])

        @pl.when(wid % workers_per_b == workers_per_b - 1)
        def _():
            zero = jnp.zeros((16,), x.dtype)

            def zrow(i, carry):
                zbuf[i // (D // 16), pl.ds((i % (D // 16)) * 16, 16)] = zero
                return carry

            lax.fori_loop(0, (_SHIFT * D) // 16, zrow, 0)
            pltpu.sync_copy(zbuf, out_hbm.at[b, pl.ds(L - _SHIFT, _SHIFT), 1])

    return span_sc(x2)


# speedup vs baseline: 1.4752x; 1.4752x over previous
import jax
import jax.numpy as jnp
from jax.experimental import pallas as pl
from jax.experimental.pallas import tpu as pltpu

_K = 16
_SHIFT = _K - 1


def _span_kernel(x_cur_ref, x_nxt_ref, out_ref, *, tl, L):
    i = pl.program_id(1)
    cur = x_cur_ref[0]
    nxt = x_nxt_ref[0]
    out_ref[0, :, 0, :] = cur
    rolled = pltpu.roll(cur, tl - _SHIFT, 0)
    out_ref[0, :, 1, :] = rolled
    nb = L // tl
    tail = jnp.where(i == nb - 1, jnp.zeros_like(nxt[:_SHIFT]), nxt[:_SHIFT])
    out_ref[0, pl.ds(tl - _SHIFT, _SHIFT), 1, :] = tail


def kernel(x):
    B, L, D = x.shape
    TL = 1024
    nb = L // TL

    out = pl.pallas_call(
        lambda a, b, o: _span_kernel(a, b, o, tl=TL, L=L),
        grid=(B, nb),
        in_specs=[
            pl.BlockSpec((1, TL, D), lambda b, i: (b, i, 0)),
            pl.BlockSpec(
                (1, 16, D),
                lambda b, i: (b, jnp.minimum((i + 1) * (TL // 16), L // 16 - 1), 0),
            ),
        ],
        out_specs=pl.BlockSpec((1, TL, 2, D), lambda b, i: (b, i, 0, 0)),
        out_shape=jax.ShapeDtypeStruct((B, L, 2, D), x.dtype),
    )(x, x)
    return out
